# baseline (speedup 1.0000x reference)
import functools

import jax
import jax.numpy as jnp
from jax import lax
from jax.experimental import pallas as pl
from jax.experimental.pallas import tpu as pltpu
from jax.experimental.pallas import tpu_sc as plsc

N, H, D, E = 4096, 1024, 1024, 8
K = 2
A = N * K
BM = 256
P = A + E * BM
NB = P // BM
TB = 256
NTB = N // TB
NC, NS, L = 2, 16, 16
NW = NC * NS
APW = A // NW
VPW = APW // L
CH = 16
TPW = N // NW
GCH = 64
NEG = -2147483647


def _s1_body(x_ref, gw_ref, ws_ref, ysh_ref, xw0_ref, xw1_ref, e0_ref, e1_ref,
             h0_ref, h1_ref):
    xb = x_ref[...]
    logits = lax.dot_general(xb, gw_ref[...], (((1,), (1,)), ((), ())),
                             preferred_element_type=jnp.float32)
    lane = lax.broadcasted_iota(jnp.int32, (TB, E), 1)
    m0 = jnp.max(logits, axis=1, keepdims=True)
    i0 = jnp.min(jnp.where(logits == m0, lane, E), axis=1)
    l2 = jnp.where(lane == i0[:, None], jnp.float32(-1e30), logits)
    m1 = jnp.max(l2, axis=1, keepdims=True)
    i1 = jnp.min(jnp.where(l2 == m1, lane, E), axis=1)
    t = jnp.exp(m1 - m0)
    w0 = 1.0 / (1.0 + t)
    w1 = t * w0
    xw0_ref[...] = (xb * w0).astype(jnp.bfloat16)
    xw1_ref[...] = (xb * w1).astype(jnp.bfloat16)
    e0_ref[0, 0, :] = i0
    e1_ref[0, 0, :] = i1
    lane128 = lax.broadcasted_iota(jnp.int32, (TB, 128), 1)
    h0_ref[0, 0, :] = jnp.sum((i0[:, None] == lane128).astype(jnp.int32), axis=0)
    h1_ref[0, 0, :] = jnp.sum((i1[:, None] == lane128).astype(jnp.int32), axis=0)
    ysh_ref[...] = jnp.dot(xb.astype(jnp.bfloat16), ws_ref[...],
                           preferred_element_type=jnp.float32)


def _s1_call(x, gate_weight, W_shared):
    return pl.pallas_call(
        _s1_body,
        grid=(NTB,),
        in_specs=[
            pl.BlockSpec((TB, H), lambda i: (i, 0)),
            pl.BlockSpec((E, H), lambda i: (0, 0)),
            pl.BlockSpec((H, D), lambda i: (0, 0)),
        ],
        out_specs=[
            pl.BlockSpec((TB, D), lambda i: (i, 0)),
            pl.BlockSpec((TB, H), lambda i: (i, 0)),
            pl.BlockSpec((TB, H), lambda i: (i, 0)),
            pl.BlockSpec((1, 1, TB), lambda i: (i, 0, 0)),
            pl.BlockSpec((1, 1, TB), lambda i: (i, 0, 0)),
            pl.BlockSpec((1, 1, 128), lambda i: (i, 0, 0)),
            pl.BlockSpec((1, 1, 128), lambda i: (i, 0, 0)),
        ],
        out_shape=[
            jax.ShapeDtypeStruct((N, D), jnp.float32),
            jax.ShapeDtypeStruct((N, H), jnp.bfloat16),
            jax.ShapeDtypeStruct((N, H), jnp.bfloat16),
            jax.ShapeDtypeStruct((NTB, 1, TB), jnp.int32),
            jax.ShapeDtypeStruct((NTB, 1, TB), jnp.int32),
            jax.ShapeDtypeStruct((NTB, 1, 128), jnp.int32),
            jax.ShapeDtypeStruct((NTB, 1, 128), jnp.int32),
        ],
    )(x, gate_weight, W_shared)


@functools.lru_cache(maxsize=None)
def _sc_mesh():
    return plsc.VectorSubcoreMesh(core_axis_name="c", subcore_axis_name="s",
                                  num_cores=NC, num_subcores=NS)


def _splat_lane(vec, lane16, e):
    return jnp.max(jnp.where(lane16 == e, vec, NEG))


def _s2_body(hist_hbm, e01_hbm, xw0_hbm, xw1_hbm,
             xs_hbm, slot_hbm, be_hbm,
             histv, ev, slots2d, idx2d, rows, bev, gsem, gsem2, ssem, ssem2):
    cid = lax.axis_index("c")
    sid = lax.axis_index("s")
    wid = sid * NC + cid
    lane16 = lax.iota(jnp.int32, L)

    pltpu.sync_copy(hist_hbm, histv)
    pltpu.sync_copy(e01_hbm.at[pl.ds(wid * APW, APW)], ev)

    totals = histv[0, 0:L]
    mybase = jnp.where(jnp.int32(0) < wid, histv[0, 0:L], 0)
    for i in range(1, NW):
        row = histv[i, 0:L]
        totals = totals + row
        mybase = mybase + jnp.where(jnp.int32(i) < wid, row, 0)
    caps = jnp.bitwise_and(totals + (BM - 1), jnp.int32(~(BM - 1)))
    cap_end = jnp.cumsum(caps)
    cap_off = cap_end - caps
    cnt = cap_off + mybase

    for v in range(VPW):
        ids = ev[pl.ds(v * L, L)]
        slot = jnp.zeros((L,), jnp.int32)
        for e in range(E):
            mask = ids == e
            incl = jnp.cumsum(jnp.where(mask, 1, 0))
            base_e = _splat_lane(cnt, lane16, e)
            slot = jnp.where(mask, base_e + incl - 1, slot)
            cnt = cnt + jnp.where(lane16 == e, jnp.max(incl), 0)
        r, c = (v * L) // GCH, (v * L) % GCH
        slots2d[r, pl.ds(c, L)] = slot
        idx2d[r, pl.ds(c, L)] = (wid % NS) * APW + v * L + lane16

    pltpu.sync_copy(slots2d, slot_hbm.at[wid])

    NCH = APW // GCH
    gsems = (gsem, gsem2)
    ssems = (ssem, ssem2)

    def _pipe(src_hbm):
        pltpu.async_copy(src_hbm.at[idx2d.at[0]], rows.at[0], gsems[0])
        for ch in range(NCH):
            p = ch % 2
            pltpu.make_async_copy(src_hbm.at[idx2d.at[ch]], rows.at[p],
                                  gsems[p]).wait()
            pltpu.async_copy(rows.at[p], xs_hbm.at[slots2d.at[ch]], ssems[p])
            if ch + 1 < NCH:
                q = (ch + 1) % 2
                if ch >= 1:
                    pltpu.make_async_copy(rows.at[q],
                                          xs_hbm.at[slots2d.at[ch - 1]],
                                          ssems[q]).wait()
                pltpu.async_copy(src_hbm.at[idx2d.at[ch + 1]], rows.at[q],
                                 gsems[q])
        for cc in (NCH - 2, NCH - 1):
            pltpu.make_async_copy(rows.at[cc % 2], xs_hbm.at[slots2d.at[cc]],
                                  ssems[cc % 2]).wait()

    @pl.when(wid < NS)
    def _():
        _pipe(xw0_hbm)

    @pl.when(wid >= NS)
    def _():
        _pipe(xw1_hbm)

    @pl.when(wid == 0)
    def _():
        for v in range(4):
            start = (lane16 + v * L) * BM
            acc = jnp.zeros((L,), jnp.int32)
            for e in range(E):
                ce = _splat_lane(cap_end, lane16, e)
                acc = acc + jnp.where(start >= ce, 1, 0)
            bev[pl.ds(v * L, L)] = jnp.minimum(acc, E - 1)
        pltpu.sync_copy(bev, be_hbm)


def _s2_call(hist, e01, xw0, xw1):
    return pl.kernel(
        _s2_body,
        out_type=[
            jax.ShapeDtypeStruct((P, H // 2), jnp.int32),
            jax.ShapeDtypeStruct((NW, APW // GCH, GCH), jnp.int32),
            jax.ShapeDtypeStruct((64,), jnp.int32),
        ],
        mesh=_sc_mesh(),
        compiler_params=pltpu.CompilerParams(needs_layout_passes=False),
        scratch_types=[
            pltpu.VMEM((NW, 128), jnp.int32),
            pltpu.VMEM((APW,), jnp.int32),
            pltpu.VMEM((APW // GCH, GCH), jnp.int32),
            pltpu.VMEM((APW // GCH, GCH), jnp.int32),
            pltpu.VMEM((2, GCH, H // 2), jnp.int32),
            pltpu.VMEM((64,), jnp.int32),
            pltpu.SemaphoreType.DMA,
            pltpu.SemaphoreType.DMA,
            pltpu.SemaphoreType.DMA,
            pltpu.SemaphoreType.DMA,
        ],
    )(hist, e01, xw0, xw1)


def _s3_body(be_ref, xs_ref, w_ref, yr_ref):
    yr_ref[...] = jnp.dot(xs_ref[...], w_ref[0], preferred_element_type=jnp.float32)


def _s3_call(blkexp, Xs, W_routed):
    return pl.pallas_call(
        _s3_body,
        grid_spec=pltpu.PrefetchScalarGridSpec(
            num_scalar_prefetch=1,
            grid=(NB,),
            in_specs=[
                pl.BlockSpec((BM, H), lambda i, be: (i, 0)),
                pl.BlockSpec((1, H, D), lambda i, be: (be[i], 0, 0)),
            ],
            out_specs=pl.BlockSpec((BM, D), lambda i, be: (i, 0)),
        ),
        out_shape=jax.ShapeDtypeStruct((P, D), jnp.float32),
    )(blkexp, Xs, W_routed)


def _s4_body(ysh_hbm, yr_hbm, slot_hbm, out_hbm,
             s0, s1, abuf, bbuf, cbuf, semA0, semA1, semB0, semB1, semC0, semC1):
    cid = lax.axis_index("c")
    sid = lax.axis_index("s")
    wid = sid * NC + cid
    tok0 = wid * TPW
    pltpu.sync_copy(slot_hbm.at[pl.ds(tok0, TPW)], s0)
    pltpu.sync_copy(slot_hbm.at[pl.ds(N + tok0, TPW)], s1)
    semA = (semA0, semA1)
    semB = (semB0, semB1)
    semC = (semC0, semC1)
    NCH2 = TPW // CH

    def descs(ch, p):
        sl = pl.ds(ch * CH, CH)
        return (
            pltpu.make_async_copy(yr_hbm.at[s0.at[sl]], abuf.at[p], semA[p]),
            pltpu.make_async_copy(yr_hbm.at[s1.at[sl]], bbuf.at[p], semB[p]),
            pltpu.make_async_copy(ysh_hbm.at[pl.ds(tok0 + ch * CH, CH)],
                                  cbuf.at[p], semC[p]),
        )

    for dsc in descs(0, 0):
        dsc.start()
    for ch in range(NCH2):
        p = ch % 2
        for dsc in descs(ch, p):
            dsc.wait()
        if ch + 1 < NCH2:
            for dsc in descs(ch + 1, 1 - p):
                dsc.start()

        def body(r, _):
            def inner(j, _):
                sl = pl.ds(j * L, L)
                abuf[p, r, sl] = abuf[p, r, sl] + bbuf[p, r, sl] + cbuf[p, r, sl]
                return 0
            return lax.fori_loop(0, D // L, inner, 0)

        lax.fori_loop(0, CH, body, 0)
        pltpu.sync_copy(abuf.at[p], out_hbm.at[pl.ds(tok0 + ch * CH, CH)])


def _s4_call(Ysh, Yr, slot01):
    return pl.kernel(
        _s4_body,
        out_type=jax.ShapeDtypeStruct((N, D), jnp.float32),
        mesh=_sc_mesh(),
        compiler_params=pltpu.CompilerParams(needs_layout_passes=False),
        scratch_types=[
            pltpu.VMEM((TPW,), jnp.int32),
            pltpu.VMEM((TPW,), jnp.int32),
            pltpu.VMEM((2, CH, D), jnp.float32),
            pltpu.VMEM((2, CH, D), jnp.float32),
            pltpu.VMEM((2, CH, D), jnp.float32),
            pltpu.SemaphoreType.DMA,
            pltpu.SemaphoreType.DMA,
            pltpu.SemaphoreType.DMA,
            pltpu.SemaphoreType.DMA,
            pltpu.SemaphoreType.DMA,
            pltpu.SemaphoreType.DMA,
        ],
    )(Ysh, Yr, slot01)


def kernel(x, gate_weight, W_routed, W_shared):
    Ysh, xw0, xw1, e0, e1, h0, h1 = _s1_call(
        x, gate_weight, W_shared.astype(jnp.bfloat16))
    e01 = jnp.concatenate([e0.reshape(N), e1.reshape(N)])
    hist = jnp.concatenate([h0.reshape(NTB, 128), h1.reshape(NTB, 128)],
                           axis=0)
    xw0i = lax.bitcast_convert_type(xw0.reshape(N, H // 2, 2), jnp.int32)
    xw1i = lax.bitcast_convert_type(xw1.reshape(N, H // 2, 2), jnp.int32)
    Xsi, slots, blkexp = _s2_call(hist, e01, xw0i, xw1i)
    Xs = lax.bitcast_convert_type(Xsi, jnp.bfloat16).reshape(P, H)
    Yr = _s3_call(blkexp, Xs, W_routed.astype(jnp.bfloat16))
    return _s4_call(Ysh, Yr, slots.reshape(A))


# speedup vs baseline: 3.1401x; 3.1401x over previous
import functools

import jax
import jax.numpy as jnp
from jax import lax
from jax.experimental import pallas as pl
from jax.experimental.pallas import tpu as pltpu
from jax.experimental.pallas import tpu_sc as plsc

N, H, D, E = 4096, 1024, 1024, 8
K = 2
A = N * K
BM = 256
P = A + E * BM
NB = P // BM
TB = 256
NTB = N // TB
NC, NS, L = 2, 16, 16
NW = NC * NS
APW = A // NW
VPW = APW // L
CH = 16
TPW = N // NW
GCH = 32
NEG = -2147483647


def _s1_body(x_ref, gw_ref, ws_ref, ysh_ref, xw0_ref, xw1_ref, e0_ref, e1_ref,
             h0_ref, h1_ref):
    xb = x_ref[...]
    logits = lax.dot_general(xb, gw_ref[...], (((1,), (1,)), ((), ())),
                             preferred_element_type=jnp.float32)
    lane = lax.broadcasted_iota(jnp.int32, (TB, E), 1)
    m0 = jnp.max(logits, axis=1, keepdims=True)
    i0 = jnp.min(jnp.where(logits == m0, lane, E), axis=1)
    l2 = jnp.where(lane == i0[:, None], jnp.float32(-1e30), logits)
    m1 = jnp.max(l2, axis=1, keepdims=True)
    i1 = jnp.min(jnp.where(l2 == m1, lane, E), axis=1)
    t = jnp.exp(m1 - m0)
    w0 = 1.0 / (1.0 + t)
    w1 = t * w0
    xw0_ref[...] = xb * w0
    xw1_ref[...] = xb * w1
    e0_ref[0, 0, :] = i0
    e1_ref[0, 0, :] = i1
    lane128 = lax.broadcasted_iota(jnp.int32, (TB, 128), 1)
    h0_ref[0, 0, :] = jnp.sum((i0[:, None] == lane128).astype(jnp.int32), axis=0)
    h1_ref[0, 0, :] = jnp.sum((i1[:, None] == lane128).astype(jnp.int32), axis=0)
    ysh_ref[...] = jnp.dot(xb.astype(jnp.bfloat16), ws_ref[...],
                           preferred_element_type=jnp.float32)


def _s1_call(x, gate_weight, W_shared):
    return pl.pallas_call(
        _s1_body,
        grid=(NTB,),
        in_specs=[
            pl.BlockSpec((TB, H), lambda i: (i, 0)),
            pl.BlockSpec((E, H), lambda i: (0, 0)),
            pl.BlockSpec((H, D), lambda i: (0, 0)),
        ],
        out_specs=[
            pl.BlockSpec((TB, D), lambda i: (i, 0)),
            pl.BlockSpec((TB, H), lambda i: (i, 0)),
            pl.BlockSpec((TB, H), lambda i: (i, 0)),
            pl.BlockSpec((1, 1, TB), lambda i: (i, 0, 0)),
            pl.BlockSpec((1, 1, TB), lambda i: (i, 0, 0)),
            pl.BlockSpec((1, 1, 128), lambda i: (i, 0, 0)),
            pl.BlockSpec((1, 1, 128), lambda i: (i, 0, 0)),
        ],
        out_shape=[
            jax.ShapeDtypeStruct((N, D), jnp.float32),
            jax.ShapeDtypeStruct((N, H), jnp.float32),
            jax.ShapeDtypeStruct((N, H), jnp.float32),
            jax.ShapeDtypeStruct((NTB, 1, TB), jnp.int32),
            jax.ShapeDtypeStruct((NTB, 1, TB), jnp.int32),
            jax.ShapeDtypeStruct((NTB, 1, 128), jnp.int32),
            jax.ShapeDtypeStruct((NTB, 1, 128), jnp.int32),
        ],
    )(x, gate_weight, W_shared)


@functools.lru_cache(maxsize=None)
def _sc_mesh():
    return plsc.VectorSubcoreMesh(core_axis_name="c", subcore_axis_name="s",
                                  num_cores=NC, num_subcores=NS)


def _splat_lane(vec, lane16, e):
    return jnp.max(jnp.where(lane16 == e, vec, NEG))


def _s2_body(hist_hbm, e01_hbm, xw0_hbm, xw1_hbm,
             xs_hbm, slot_hbm, be_hbm,
             histv, ev, slots2d, idx2d, rows, bev, gsem, gsem2, ssem, ssem2):
    cid = lax.axis_index("c")
    sid = lax.axis_index("s")
    wid = sid * NC + cid
    lane16 = lax.iota(jnp.int32, L)

    pltpu.sync_copy(hist_hbm, histv)
    pltpu.sync_copy(e01_hbm.at[pl.ds(wid * APW, APW)], ev)

    totals = histv[0, 0:L]
    mybase = jnp.where(jnp.int32(0) < wid, histv[0, 0:L], 0)
    for i in range(1, NW):
        row = histv[i, 0:L]
        totals = totals + row
        mybase = mybase + jnp.where(jnp.int32(i) < wid, row, 0)
    caps = jnp.bitwise_and(totals + (BM - 1), jnp.int32(~(BM - 1)))
    cap_end = jnp.cumsum(caps)
    cap_off = cap_end - caps
    cnt = cap_off + mybase

    for v in range(VPW):
        ids = ev[pl.ds(v * L, L)]
        slot = jnp.zeros((L,), jnp.int32)
        for e in range(E):
            mask = ids == e
            incl = jnp.cumsum(jnp.where(mask, 1, 0))
            base_e = _splat_lane(cnt, lane16, e)
            slot = jnp.where(mask, base_e + incl - 1, slot)
            cnt = cnt + jnp.where(lane16 == e, jnp.max(incl), 0)
        r, c = (v * L) // GCH, (v * L) % GCH
        slots2d[r, pl.ds(c, L)] = slot
        idx2d[r, pl.ds(c, L)] = (wid % NS) * APW + v * L + lane16

    pltpu.sync_copy(slots2d, slot_hbm.at[wid])

    NCH = APW // GCH
    gsems = (gsem, gsem2)
    ssems = (ssem, ssem2)

    def _pipe(src_hbm):
        pltpu.async_copy(src_hbm.at[idx2d.at[0]], rows.at[0], gsems[0])
        for ch in range(NCH):
            p = ch % 2
            pltpu.make_async_copy(src_hbm.at[idx2d.at[ch]], rows.at[p],
                                  gsems[p]).wait()
            pltpu.async_copy(rows.at[p], xs_hbm.at[slots2d.at[ch]], ssems[p])
            if ch + 1 < NCH:
                q = (ch + 1) % 2
                if ch >= 1:
                    pltpu.make_async_copy(rows.at[q],
                                          xs_hbm.at[slots2d.at[ch - 1]],
                                          ssems[q]).wait()
                pltpu.async_copy(src_hbm.at[idx2d.at[ch + 1]], rows.at[q],
                                 gsems[q])
        for cc in (NCH - 2, NCH - 1):
            pltpu.make_async_copy(rows.at[cc % 2], xs_hbm.at[slots2d.at[cc]],
                                  ssems[cc % 2]).wait()

    @pl.when(wid < NS)
    def _():
        _pipe(xw0_hbm)

    @pl.when(wid >= NS)
    def _():
        _pipe(xw1_hbm)

    @pl.when(wid == 0)
    def _():
        for v in range(4):
            start = (lane16 + v * L) * BM
            acc = jnp.zeros((L,), jnp.int32)
            for e in range(E):
                ce = _splat_lane(cap_end, lane16, e)
                acc = acc + jnp.where(start >= ce, 1, 0)
            bev[pl.ds(v * L, L)] = jnp.minimum(acc, E - 1)
        pltpu.sync_copy(bev, be_hbm)


def _s2_call(hist, e01, xw0, xw1):
    return pl.kernel(
        _s2_body,
        out_type=[
            jax.ShapeDtypeStruct((P, H), jnp.float32),
            jax.ShapeDtypeStruct((NW, APW // GCH, GCH), jnp.int32),
            jax.ShapeDtypeStruct((64,), jnp.int32),
        ],
        mesh=_sc_mesh(),
        compiler_params=pltpu.CompilerParams(needs_layout_passes=False),
        scratch_types=[
            pltpu.VMEM((NW, 128), jnp.int32),
            pltpu.VMEM((APW,), jnp.int32),
            pltpu.VMEM((APW // GCH, GCH), jnp.int32),
            pltpu.VMEM((APW // GCH, GCH), jnp.int32),
            pltpu.VMEM((2, GCH, H), jnp.float32),
            pltpu.VMEM((64,), jnp.int32),
            pltpu.SemaphoreType.DMA,
            pltpu.SemaphoreType.DMA,
            pltpu.SemaphoreType.DMA,
            pltpu.SemaphoreType.DMA,
        ],
    )(hist, e01, xw0, xw1)


def _s3_body(be_ref, xs_ref, w_ref, yr_ref):
    yr_ref[...] = jnp.dot(xs_ref[...].astype(jnp.bfloat16), w_ref[0],
                          preferred_element_type=jnp.float32)


def _s3_call(blkexp, Xs, W_routed):
    return pl.pallas_call(
        _s3_body,
        grid_spec=pltpu.PrefetchScalarGridSpec(
            num_scalar_prefetch=1,
            grid=(NB,),
            in_specs=[
                pl.BlockSpec((BM, H), lambda i, be: (i, 0)),
                pl.BlockSpec((1, H, D), lambda i, be: (be[i], 0, 0)),
            ],
            out_specs=pl.BlockSpec((BM, D), lambda i, be: (i, 0)),
        ),
        out_shape=jax.ShapeDtypeStruct((P, D), jnp.float32),
    )(blkexp, Xs, W_routed)


def _s4_body(ysh_hbm, yr_hbm, slot_hbm, out_hbm,
             s0, s1, abuf, bbuf, cbuf, semA0, semA1, semB0, semB1, semC0, semC1):
    cid = lax.axis_index("c")
    sid = lax.axis_index("s")
    wid = sid * NC + cid
    tok0 = wid * TPW
    pltpu.sync_copy(slot_hbm.at[pl.ds(tok0, TPW)], s0)
    pltpu.sync_copy(slot_hbm.at[pl.ds(N + tok0, TPW)], s1)
    semA = (semA0, semA1)
    semB = (semB0, semB1)
    semC = (semC0, semC1)
    NCH2 = TPW // CH

    def descs(ch, p):
        sl = pl.ds(ch * CH, CH)
        return (
            pltpu.make_async_copy(yr_hbm.at[s0.at[sl]], abuf.at[p], semA[p]),
            pltpu.make_async_copy(yr_hbm.at[s1.at[sl]], bbuf.at[p], semB[p]),
            pltpu.make_async_copy(ysh_hbm.at[pl.ds(tok0 + ch * CH, CH)],
                                  cbuf.at[p], semC[p]),
        )

    for dsc in descs(0, 0):
        dsc.start()
    for ch in range(NCH2):
        p = ch % 2
        for dsc in descs(ch, p):
            dsc.wait()
        if ch + 1 < NCH2:
            for dsc in descs(ch + 1, 1 - p):
                dsc.start()

        @plsc.parallel_loop(0, CH * (D // L), unroll=8)
        def _(i):
            r = lax.shift_right_logical(i, 6)
            c = lax.shift_left(jnp.bitwise_and(i, (D // L) - 1), 4)
            sl = pl.ds(c, L)
            abuf[p, r, sl] = abuf[p, r, sl] + bbuf[p, r, sl] + cbuf[p, r, sl]
        pltpu.sync_copy(abuf.at[p], out_hbm.at[pl.ds(tok0 + ch * CH, CH)])


def _s4_call(Ysh, Yr, slot01):
    return pl.kernel(
        _s4_body,
        out_type=jax.ShapeDtypeStruct((N, D), jnp.float32),
        mesh=_sc_mesh(),
        compiler_params=pltpu.CompilerParams(needs_layout_passes=False),
        scratch_types=[
            pltpu.VMEM((TPW,), jnp.int32),
            pltpu.VMEM((TPW,), jnp.int32),
            pltpu.VMEM((2, CH, D), jnp.float32),
            pltpu.VMEM((2, CH, D), jnp.float32),
            pltpu.VMEM((2, CH, D), jnp.float32),
            pltpu.SemaphoreType.DMA,
            pltpu.SemaphoreType.DMA,
            pltpu.SemaphoreType.DMA,
            pltpu.SemaphoreType.DMA,
            pltpu.SemaphoreType.DMA,
            pltpu.SemaphoreType.DMA,
        ],
    )(Ysh, Yr, slot01)


def kernel(x, gate_weight, W_routed, W_shared):
    Ysh, xw0, xw1, e0, e1, h0, h1 = _s1_call(
        x, gate_weight, W_shared.astype(jnp.bfloat16))
    e01 = jnp.concatenate([e0.reshape(N), e1.reshape(N)])
    hist = jnp.concatenate([h0.reshape(NTB, 128), h1.reshape(NTB, 128)],
                           axis=0)
    Xs, slots, blkexp = _s2_call(hist, e01, xw0, xw1)
    Yr = _s3_call(blkexp, Xs, W_routed.astype(jnp.bfloat16))
    return _s4_call(Ysh, Yr, slots.reshape(A))


# speedup vs baseline: 3.2566x; 1.0371x over previous
import functools

import jax
import jax.numpy as jnp
from jax import lax
from jax.experimental import pallas as pl
from jax.experimental.pallas import tpu as pltpu
from jax.experimental.pallas import tpu_sc as plsc

N, H, D, E = 4096, 1024, 1024, 8
K = 2
A = N * K
BM = 256
P = A + E * BM
NB = P // BM
TB = 256
NTB = N // TB
NC, NS, L = 2, 16, 16
NW = NC * NS
APW = A // NW
VPW = APW // L
CH = 16
TPW = N // NW
GCH = 32
NEG = -2147483647


def _s1_body(x_ref, gw_ref, ws_ref, ysh_ref, xw0_ref, xw1_ref, e01_ref,
             hh_ref):
    xb = x_ref[...]
    logits = lax.dot_general(xb, gw_ref[...], (((1,), (1,)), ((), ())),
                             preferred_element_type=jnp.float32)
    lane = lax.broadcasted_iota(jnp.int32, (TB, E), 1)
    m0 = jnp.max(logits, axis=1, keepdims=True)
    i0 = jnp.min(jnp.where(logits == m0, lane, E), axis=1)
    l2 = jnp.where(lane == i0[:, None], jnp.float32(-1e30), logits)
    m1 = jnp.max(l2, axis=1, keepdims=True)
    i1 = jnp.min(jnp.where(l2 == m1, lane, E), axis=1)
    t = jnp.exp(m1 - m0)
    w0 = 1.0 / (1.0 + t)
    w1 = t * w0
    xw0_ref[...] = xb * w0
    xw1_ref[...] = xb * w1
    e01_ref[0, 0, 0, :] = i0
    e01_ref[1, 0, 0, :] = i1
    lane128 = lax.broadcasted_iota(jnp.int32, (TB, 128), 1)
    hh_ref[0, 0, 0, :] = jnp.sum((i0[:, None] == lane128).astype(jnp.int32), axis=0)
    hh_ref[1, 0, 0, :] = jnp.sum((i1[:, None] == lane128).astype(jnp.int32), axis=0)
    ysh_ref[...] = jnp.dot(xb.astype(jnp.bfloat16), ws_ref[...],
                           preferred_element_type=jnp.float32)


def _s1_call(x, gate_weight, W_shared):
    return pl.pallas_call(
        _s1_body,
        grid=(NTB,),
        in_specs=[
            pl.BlockSpec((TB, H), lambda i: (i, 0)),
            pl.BlockSpec((E, H), lambda i: (0, 0)),
            pl.BlockSpec((H, D), lambda i: (0, 0)),
        ],
        out_specs=[
            pl.BlockSpec((TB, D), lambda i: (i, 0)),
            pl.BlockSpec((TB, H), lambda i: (i, 0)),
            pl.BlockSpec((TB, H), lambda i: (i, 0)),
            pl.BlockSpec((2, 1, 1, TB), lambda i: (0, i, 0, 0)),
            pl.BlockSpec((2, 1, 1, 128), lambda i: (0, i, 0, 0)),
        ],
        out_shape=[
            jax.ShapeDtypeStruct((N, D), jnp.float32),
            jax.ShapeDtypeStruct((N, H), jnp.float32),
            jax.ShapeDtypeStruct((N, H), jnp.float32),
            jax.ShapeDtypeStruct((K, NTB, 1, TB), jnp.int32),
            jax.ShapeDtypeStruct((K, NTB, 1, 128), jnp.int32),
        ],
    )(x, gate_weight, W_shared)


@functools.lru_cache(maxsize=None)
def _sc_mesh():
    return plsc.VectorSubcoreMesh(core_axis_name="c", subcore_axis_name="s",
                                  num_cores=NC, num_subcores=NS)


def _splat_lane(vec, lane16, e):
    return jnp.max(jnp.where(lane16 == e, vec, NEG))


def _s2_body(hist_hbm, e01_hbm, xw0_hbm, xw1_hbm,
             xs_hbm, slot_hbm, be_hbm,
             histv, ev, slots2d, idx2d, rows, bev, gsem, gsem2, ssem, ssem2):
    cid = lax.axis_index("c")
    sid = lax.axis_index("s")
    wid = sid * NC + cid
    lane16 = lax.iota(jnp.int32, L)

    pltpu.sync_copy(hist_hbm, histv)
    pltpu.sync_copy(e01_hbm.at[pl.ds(wid * APW, APW)], ev)

    totals = histv[0, 0:L]
    mybase = jnp.where(jnp.int32(0) < wid, histv[0, 0:L], 0)
    for i in range(1, NW):
        row = histv[i, 0:L]
        totals = totals + row
        mybase = mybase + jnp.where(jnp.int32(i) < wid, row, 0)
    caps = jnp.bitwise_and(totals + (BM - 1), jnp.int32(~(BM - 1)))
    cap_end = jnp.cumsum(caps)
    cap_off = cap_end - caps
    cnt = cap_off + mybase

    for v in range(VPW):
        ids = ev[pl.ds(v * L, L)]
        slot = jnp.zeros((L,), jnp.int32)
        for e in range(E):
            mask = ids == e
            incl = jnp.cumsum(jnp.where(mask, 1, 0))
            base_e = _splat_lane(cnt, lane16, e)
            slot = jnp.where(mask, base_e + incl - 1, slot)
            cnt = cnt + jnp.where(lane16 == e, jnp.max(incl), 0)
        r, c = (v * L) // GCH, (v * L) % GCH
        slots2d[r, pl.ds(c, L)] = slot
        idx2d[r, pl.ds(c, L)] = (wid % NS) * APW + v * L + lane16

    pltpu.sync_copy(slots2d, slot_hbm.at[wid])

    NCH = APW // GCH
    gsems = (gsem, gsem2)
    ssems = (ssem, ssem2)

    def _pipe(src_hbm):
        pltpu.async_copy(src_hbm.at[idx2d.at[0]], rows.at[0], gsems[0])
        for ch in range(NCH):
            p = ch % 2
            pltpu.make_async_copy(src_hbm.at[idx2d.at[ch]], rows.at[p],
                                  gsems[p]).wait()
            pltpu.async_copy(rows.at[p], xs_hbm.at[slots2d.at[ch]], ssems[p])
            if ch + 1 < NCH:
                q = (ch + 1) % 2
                if ch >= 1:
                    pltpu.make_async_copy(rows.at[q],
                                          xs_hbm.at[slots2d.at[ch - 1]],
                                          ssems[q]).wait()
                pltpu.async_copy(src_hbm.at[idx2d.at[ch + 1]], rows.at[q],
                                 gsems[q])
        for cc in (NCH - 2, NCH - 1):
            pltpu.make_async_copy(rows.at[cc % 2], xs_hbm.at[slots2d.at[cc]],
                                  ssems[cc % 2]).wait()

    @pl.when(wid < NS)
    def _():
        _pipe(xw0_hbm)

    @pl.when(wid >= NS)
    def _():
        _pipe(xw1_hbm)

    @pl.when(wid == 0)
    def _():
        for v in range(4):
            start = (lane16 + v * L) * BM
            acc = jnp.zeros((L,), jnp.int32)
            for e in range(E):
                ce = _splat_lane(cap_end, lane16, e)
                acc = acc + jnp.where(start >= ce, 1, 0)
            bev[pl.ds(v * L, L)] = jnp.minimum(acc, E - 1)
        pltpu.sync_copy(bev, be_hbm)


def _s2_call(hist, e01, xw0, xw1):
    return pl.kernel(
        _s2_body,
        out_type=[
            jax.ShapeDtypeStruct((P, H), jnp.float32),
            jax.ShapeDtypeStruct((NW, APW // GCH, GCH), jnp.int32),
            jax.ShapeDtypeStruct((64,), jnp.int32),
        ],
        mesh=_sc_mesh(),
        compiler_params=pltpu.CompilerParams(needs_layout_passes=False),
        scratch_types=[
            pltpu.VMEM((NW, 128), jnp.int32),
            pltpu.VMEM((APW,), jnp.int32),
            pltpu.VMEM((APW // GCH, GCH), jnp.int32),
            pltpu.VMEM((APW // GCH, GCH), jnp.int32),
            pltpu.VMEM((2, GCH, H), jnp.float32),
            pltpu.VMEM((64,), jnp.int32),
            pltpu.SemaphoreType.DMA,
            pltpu.SemaphoreType.DMA,
            pltpu.SemaphoreType.DMA,
            pltpu.SemaphoreType.DMA,
        ],
    )(hist, e01, xw0, xw1)


def _s3_body(be_ref, xs_ref, w_ref, yr_ref, wc_ref, last_ref):
    i = pl.program_id(0)
    e = be_ref[i]

    @pl.when((i == 0) | (e != last_ref[0]))
    def _():
        wc_ref[...] = w_ref[0].astype(jnp.bfloat16)

    last_ref[0] = e
    yr_ref[...] = jnp.dot(xs_ref[...].astype(jnp.bfloat16), wc_ref[...],
                          preferred_element_type=jnp.float32)


def _s3_call(blkexp, Xs, W_routed):
    return pl.pallas_call(
        _s3_body,
        grid_spec=pltpu.PrefetchScalarGridSpec(
            num_scalar_prefetch=1,
            grid=(NB,),
            in_specs=[
                pl.BlockSpec((BM, H), lambda i, be: (i, 0)),
                pl.BlockSpec((1, H, D), lambda i, be: (be[i], 0, 0)),
            ],
            out_specs=pl.BlockSpec((BM, D), lambda i, be: (i, 0)),
            scratch_shapes=[
                pltpu.VMEM((H, D), jnp.bfloat16),
                pltpu.SMEM((1,), jnp.int32),
            ],
        ),
        out_shape=jax.ShapeDtypeStruct((P, D), jnp.float32),
    )(blkexp, Xs, W_routed)


def _s4_body(ysh_hbm, yr_hbm, slot_hbm, out_hbm,
             s0, s1, abuf, bbuf, cbuf, semA0, semA1, semB0, semB1, semC0, semC1):
    cid = lax.axis_index("c")
    sid = lax.axis_index("s")
    wid = sid * NC + cid
    tok0 = wid * TPW
    pltpu.sync_copy(slot_hbm.at[pl.ds(tok0, TPW)], s0)
    pltpu.sync_copy(slot_hbm.at[pl.ds(N + tok0, TPW)], s1)
    semA = (semA0, semA1)
    semB = (semB0, semB1)
    semC = (semC0, semC1)
    NCH2 = TPW // CH

    def descs(ch, p):
        sl = pl.ds(ch * CH, CH)
        return (
            pltpu.make_async_copy(yr_hbm.at[s0.at[sl]], abuf.at[p], semA[p]),
            pltpu.make_async_copy(yr_hbm.at[s1.at[sl]], bbuf.at[p], semB[p]),
            pltpu.make_async_copy(ysh_hbm.at[pl.ds(tok0 + ch * CH, CH)],
                                  cbuf.at[p], semC[p]),
        )

    for dsc in descs(0, 0):
        dsc.start()
    for ch in range(NCH2):
        p = ch % 2
        for dsc in descs(ch, p):
            dsc.wait()
        if ch + 1 < NCH2:
            for dsc in descs(ch + 1, 1 - p):
                dsc.start()

        @plsc.parallel_loop(0, CH * (D // L), unroll=8)
        def _(i):
            r = lax.shift_right_logical(i, 6)
            c = lax.shift_left(jnp.bitwise_and(i, (D // L) - 1), 4)
            sl = pl.ds(c, L)
            abuf[p, r, sl] = abuf[p, r, sl] + bbuf[p, r, sl] + cbuf[p, r, sl]
        pltpu.sync_copy(abuf.at[p], out_hbm.at[pl.ds(tok0 + ch * CH, CH)])


def _s4_call(Ysh, Yr, slot01):
    return pl.kernel(
        _s4_body,
        out_type=jax.ShapeDtypeStruct((N, D), jnp.float32),
        mesh=_sc_mesh(),
        compiler_params=pltpu.CompilerParams(needs_layout_passes=False),
        scratch_types=[
            pltpu.VMEM((TPW,), jnp.int32),
            pltpu.VMEM((TPW,), jnp.int32),
            pltpu.VMEM((2, CH, D), jnp.float32),
            pltpu.VMEM((2, CH, D), jnp.float32),
            pltpu.VMEM((2, CH, D), jnp.float32),
            pltpu.SemaphoreType.DMA,
            pltpu.SemaphoreType.DMA,
            pltpu.SemaphoreType.DMA,
            pltpu.SemaphoreType.DMA,
            pltpu.SemaphoreType.DMA,
            pltpu.SemaphoreType.DMA,
        ],
    )(Ysh, Yr, slot01)


def kernel(x, gate_weight, W_routed, W_shared):
    Ysh, xw0, xw1, e01, hh = _s1_call(
        x, gate_weight, W_shared.astype(jnp.bfloat16))
    Xs, slots, blkexp = _s2_call(hh.reshape(NW, 128), e01.reshape(A), xw0, xw1)
    Yr = _s3_call(blkexp, Xs, W_routed)
    return _s4_call(Ysh, Yr, slots.reshape(A))


# speedup vs baseline: 3.2946x; 1.0117x over previous
import functools

import jax
import jax.numpy as jnp
from jax import lax
from jax.experimental import pallas as pl
from jax.experimental.pallas import tpu as pltpu
from jax.experimental.pallas import tpu_sc as plsc

N, H, D, E = 4096, 1024, 1024, 8
K = 2
A = N * K
BM = 256
P = A + E * BM
NB = P // BM
TB = 256
NTB = N // TB
NC, NS, L = 2, 16, 16
NW = NC * NS
APW = A // NW
VPW = APW // L
CH = 16
TPW = N // NW
GCH = 32
NEG = -2147483647


def _s1_body(x_ref, gw_ref, ws_ref, ysh_ref, xw0_ref, xw1_ref, e01_ref,
             hh_ref):
    xb = x_ref[...]
    logits = lax.dot_general(xb, gw_ref[...], (((1,), (1,)), ((), ())),
                             preferred_element_type=jnp.float32)
    lane = lax.broadcasted_iota(jnp.int32, (TB, E), 1)
    m0 = jnp.max(logits, axis=1, keepdims=True)
    i0 = jnp.min(jnp.where(logits == m0, lane, E), axis=1)
    l2 = jnp.where(lane == i0[:, None], jnp.float32(-1e30), logits)
    m1 = jnp.max(l2, axis=1, keepdims=True)
    i1 = jnp.min(jnp.where(l2 == m1, lane, E), axis=1)
    t = jnp.exp(m1 - m0)
    w0 = 1.0 / (1.0 + t)
    w1 = t * w0
    xw0_ref[...] = xb * w0
    xw1_ref[...] = xb * w1
    e01_ref[0, 0, 0, :] = i0
    e01_ref[1, 0, 0, :] = i1
    lane128 = lax.broadcasted_iota(jnp.int32, (TB, 128), 1)
    hh_ref[0, 0, 0, :] = jnp.sum((i0[:, None] == lane128).astype(jnp.int32), axis=0)
    hh_ref[1, 0, 0, :] = jnp.sum((i1[:, None] == lane128).astype(jnp.int32), axis=0)
    ysh_ref[...] = jnp.dot(xb.astype(jnp.bfloat16), ws_ref[...],
                           preferred_element_type=jnp.float32)


def _s1_call(x, gate_weight, W_shared):
    return pl.pallas_call(
        _s1_body,
        grid=(NTB,),
        in_specs=[
            pl.BlockSpec((TB, H), lambda i: (i, 0)),
            pl.BlockSpec((E, H), lambda i: (0, 0)),
            pl.BlockSpec((H, D), lambda i: (0, 0)),
        ],
        out_specs=[
            pl.BlockSpec((TB, D), lambda i: (i, 0)),
            pl.BlockSpec((TB, H), lambda i: (i, 0)),
            pl.BlockSpec((TB, H), lambda i: (i, 0)),
            pl.BlockSpec((2, 1, 1, TB), lambda i: (0, i, 0, 0)),
            pl.BlockSpec((2, 1, 1, 128), lambda i: (0, i, 0, 0)),
        ],
        out_shape=[
            jax.ShapeDtypeStruct((N, D), jnp.float32),
            jax.ShapeDtypeStruct((N, H), jnp.float32),
            jax.ShapeDtypeStruct((N, H), jnp.float32),
            jax.ShapeDtypeStruct((K, NTB, 1, TB), jnp.int32),
            jax.ShapeDtypeStruct((K, NTB, 1, 128), jnp.int32),
        ],
    )(x, gate_weight, W_shared)


@functools.lru_cache(maxsize=None)
def _sc_mesh():
    return plsc.VectorSubcoreMesh(core_axis_name="c", subcore_axis_name="s",
                                  num_cores=NC, num_subcores=NS)


def _splat_lane(vec, lane16, e):
    return jnp.max(jnp.where(lane16 == e, vec, NEG))


def _s2_body(hist_hbm, e01_hbm, xw0_hbm, xw1_hbm,
             xs_hbm, slot0_hbm, slot1_hbm, be_hbm,
             histv, ev, slots2d, slotsl, idx2d, rows, bev,
             gsem, gsem2, ssem, ssem2):
    cid = lax.axis_index("c")
    sid = lax.axis_index("s")
    wid = sid * NC + cid
    lane16 = lax.iota(jnp.int32, L)

    pltpu.sync_copy(hist_hbm, histv)
    pltpu.sync_copy(e01_hbm.at[wid // NS, wid % NS, 0], ev)

    totals = histv[0, 0, 0, 0:L]
    mybase = jnp.where(jnp.int32(0) < wid, histv[0, 0, 0, 0:L], 0)
    for i in range(1, NW):
        row = histv[i // NS, i % NS, 0, 0:L]
        totals = totals + row
        mybase = mybase + jnp.where(jnp.int32(i) < wid, row, 0)
    caps = jnp.bitwise_and(totals + (BM - 1), jnp.int32(~(BM - 1)))
    cap_end = jnp.cumsum(caps)
    cap_off = cap_end - caps
    cnt = cap_off + mybase

    for v in range(VPW):
        ids = ev[pl.ds(v * L, L)]
        slot = jnp.zeros((L,), jnp.int32)
        for e in range(E):
            mask = ids == e
            incl = jnp.cumsum(jnp.where(mask, 1, 0))
            base_e = _splat_lane(cnt, lane16, e)
            slot = jnp.where(mask, base_e + incl - 1, slot)
            cnt = cnt + jnp.where(lane16 == e, jnp.max(incl), 0)
        r, c = (v * L) // GCH, (v * L) % GCH
        slots2d[r, pl.ds(c, L)] = slot
        slotsl[pl.ds(v * L, L)] = slot
        idx2d[r, pl.ds(c, L)] = (wid % NS) * APW + v * L + lane16

    @pl.when(wid < NS)
    def _():
        pltpu.sync_copy(slotsl, slot0_hbm.at[pl.ds((wid % NS) * APW, APW)])

    @pl.when(wid >= NS)
    def _():
        pltpu.sync_copy(slotsl, slot1_hbm.at[pl.ds((wid % NS) * APW, APW)])

    NCH = APW // GCH
    gsems = (gsem, gsem2)
    ssems = (ssem, ssem2)

    def _pipe(src_hbm):
        pltpu.async_copy(src_hbm.at[idx2d.at[0]], rows.at[0], gsems[0])
        for ch in range(NCH):
            p = ch % 2
            pltpu.make_async_copy(src_hbm.at[idx2d.at[ch]], rows.at[p],
                                  gsems[p]).wait()
            pltpu.async_copy(rows.at[p], xs_hbm.at[slots2d.at[ch]], ssems[p])
            if ch + 1 < NCH:
                q = (ch + 1) % 2
                if ch >= 1:
                    pltpu.make_async_copy(rows.at[q],
                                          xs_hbm.at[slots2d.at[ch - 1]],
                                          ssems[q]).wait()
                pltpu.async_copy(src_hbm.at[idx2d.at[ch + 1]], rows.at[q],
                                 gsems[q])
        for cc in (NCH - 2, NCH - 1):
            pltpu.make_async_copy(rows.at[cc % 2], xs_hbm.at[slots2d.at[cc]],
                                  ssems[cc % 2]).wait()

    @pl.when(wid < NS)
    def _():
        _pipe(xw0_hbm)

    @pl.when(wid >= NS)
    def _():
        _pipe(xw1_hbm)

    @pl.when(wid == 0)
    def _():
        for v in range(4):
            start = (lane16 + v * L) * BM
            acc = jnp.zeros((L,), jnp.int32)
            for e in range(E):
                ce = _splat_lane(cap_end, lane16, e)
                acc = acc + jnp.where(start >= ce, 1, 0)
            bev[pl.ds(v * L, L)] = jnp.minimum(acc, E - 1)
        pltpu.sync_copy(bev, be_hbm)


def _s2_call(hist, e01, xw0, xw1):
    return pl.kernel(
        _s2_body,
        out_type=[
            jax.ShapeDtypeStruct((P, H), jnp.float32),
            jax.ShapeDtypeStruct((N,), jnp.int32),
            jax.ShapeDtypeStruct((N,), jnp.int32),
            jax.ShapeDtypeStruct((64,), jnp.int32),
        ],
        mesh=_sc_mesh(),
        compiler_params=pltpu.CompilerParams(needs_layout_passes=False),
        scratch_types=[
            pltpu.VMEM((K, NTB, 1, 128), jnp.int32),
            pltpu.VMEM((APW,), jnp.int32),
            pltpu.VMEM((APW // GCH, GCH), jnp.int32),
            pltpu.VMEM((APW,), jnp.int32),
            pltpu.VMEM((APW // GCH, GCH), jnp.int32),
            pltpu.VMEM((2, GCH, H), jnp.float32),
            pltpu.VMEM((64,), jnp.int32),
            pltpu.SemaphoreType.DMA,
            pltpu.SemaphoreType.DMA,
            pltpu.SemaphoreType.DMA,
            pltpu.SemaphoreType.DMA,
        ],
    )(hist, e01, xw0, xw1)


def _s3_body(be_ref, xs_ref, w_ref, yr_ref, wc_ref, last_ref):
    i = pl.program_id(0)
    e = be_ref[i]

    @pl.when((i == 0) | (e != last_ref[0]))
    def _():
        wc_ref[...] = w_ref[0].astype(jnp.bfloat16)

    last_ref[0] = e
    yr_ref[...] = jnp.dot(xs_ref[...].astype(jnp.bfloat16), wc_ref[...],
                          preferred_element_type=jnp.float32)


def _s3_call(blkexp, Xs, W_routed):
    return pl.pallas_call(
        _s3_body,
        grid_spec=pltpu.PrefetchScalarGridSpec(
            num_scalar_prefetch=1,
            grid=(NB,),
            in_specs=[
                pl.BlockSpec((BM, H), lambda i, be: (i, 0)),
                pl.BlockSpec((1, H, D), lambda i, be: (be[i], 0, 0)),
            ],
            out_specs=pl.BlockSpec((BM, D), lambda i, be: (i, 0)),
            scratch_shapes=[
                pltpu.VMEM((H, D), jnp.bfloat16),
                pltpu.SMEM((1,), jnp.int32),
            ],
        ),
        out_shape=jax.ShapeDtypeStruct((P, D), jnp.float32),
    )(blkexp, Xs, W_routed)


def _s4_body(ysh_hbm, yr_hbm, slot0_hbm, slot1_hbm, out_hbm,
             s0, s1, abuf, bbuf, cbuf, semA0, semA1, semB0, semB1, semC0, semC1):
    cid = lax.axis_index("c")
    sid = lax.axis_index("s")
    wid = sid * NC + cid
    tok0 = wid * TPW
    pltpu.sync_copy(slot0_hbm.at[pl.ds(tok0, TPW)], s0)
    pltpu.sync_copy(slot1_hbm.at[pl.ds(tok0, TPW)], s1)
    semA = (semA0, semA1)
    semB = (semB0, semB1)
    semC = (semC0, semC1)
    NCH2 = TPW // CH

    def descs(ch, p):
        sl = pl.ds(ch * CH, CH)
        return (
            pltpu.make_async_copy(yr_hbm.at[s0.at[sl]], abuf.at[p], semA[p]),
            pltpu.make_async_copy(yr_hbm.at[s1.at[sl]], bbuf.at[p], semB[p]),
            pltpu.make_async_copy(ysh_hbm.at[pl.ds(tok0 + ch * CH, CH)],
                                  cbuf.at[p], semC[p]),
        )

    for dsc in descs(0, 0):
        dsc.start()
    for ch in range(NCH2):
        p = ch % 2
        for dsc in descs(ch, p):
            dsc.wait()
        if ch + 1 < NCH2:
            for dsc in descs(ch + 1, 1 - p):
                dsc.start()

        @plsc.parallel_loop(0, CH * (D // L), unroll=8)
        def _(i):
            r = lax.shift_right_logical(i, 6)
            c = lax.shift_left(jnp.bitwise_and(i, (D // L) - 1), 4)
            sl = pl.ds(c, L)
            abuf[p, r, sl] = abuf[p, r, sl] + bbuf[p, r, sl] + cbuf[p, r, sl]
        pltpu.sync_copy(abuf.at[p], out_hbm.at[pl.ds(tok0 + ch * CH, CH)])


def _s4_call(Ysh, Yr, slot0, slot1):
    return pl.kernel(
        _s4_body,
        out_type=jax.ShapeDtypeStruct((N, D), jnp.float32),
        mesh=_sc_mesh(),
        compiler_params=pltpu.CompilerParams(needs_layout_passes=False),
        scratch_types=[
            pltpu.VMEM((TPW,), jnp.int32),
            pltpu.VMEM((TPW,), jnp.int32),
            pltpu.VMEM((2, CH, D), jnp.float32),
            pltpu.VMEM((2, CH, D), jnp.float32),
            pltpu.VMEM((2, CH, D), jnp.float32),
            pltpu.SemaphoreType.DMA,
            pltpu.SemaphoreType.DMA,
            pltpu.SemaphoreType.DMA,
            pltpu.SemaphoreType.DMA,
            pltpu.SemaphoreType.DMA,
            pltpu.SemaphoreType.DMA,
        ],
    )(Ysh, Yr, slot0, slot1)


def kernel(x, gate_weight, W_routed, W_shared):
    Ysh, xw0, xw1, e01, hh = _s1_call(
        x, gate_weight, W_shared.astype(jnp.bfloat16))
    Xs, slot0, slot1, blkexp = _s2_call(hh, e01, xw0, xw1)
    Yr = _s3_call(blkexp, Xs, W_routed)
    return _s4_call(Ysh, Yr, slot0, slot1)
